# speedup vs baseline: 2.1502x; 2.1502x over previous
import functools

import jax
import jax.numpy as jnp
from jax import lax
from jax.experimental import pallas as pl
from jax.experimental.pallas import tpu as pltpu
from jax.experimental.pallas import tpu_sc as plsc

HIDDEN = 256
OUT_DIM = 384
BATCH = 16384
NUM_LANES = 16
LANE_GROUPS = HIDDEN // NUM_LANES

NC = 2
NS = 16
NW = NC * NS
B_PER_W = BATCH // NW
CHUNK = 128
NCHUNK = B_PER_W // CHUNK


def _sc_partial_sums(head_idx, rel_idx, node_emb, rel_emb):
  mesh = plsc.VectorSubcoreMesh(core_axis_name="c", subcore_axis_name="s")

  @functools.partial(
      pl.kernel,
      out_type=jax.ShapeDtypeStruct((NW, HIDDEN), jnp.float32),
      mesh=mesh,
      scratch_types=[
          pltpu.VMEM((NCHUNK, CHUNK), jnp.int32),
          pltpu.VMEM((NCHUNK, CHUNK), jnp.int32),
          pltpu.VMEM((CHUNK, HIDDEN), jnp.float32),
          pltpu.VMEM((CHUNK, HIDDEN), jnp.float32),
          pltpu.VMEM((HIDDEN,), jnp.float32),
          pltpu.SemaphoreType.DMA,
          pltpu.SemaphoreType.DMA,
      ],
  )
  def sc_kernel(head_hbm, rel_hbm, node_hbm, relemb_hbm, out_hbm,
                hidx_v, ridx_v, buf_a, buf_b, acc_v, sem_a, sem_b):
    wid = lax.axis_index("s") * NC + lax.axis_index("c")
    pltpu.sync_copy(head_hbm.at[wid], hidx_v)
    pltpu.sync_copy(rel_hbm.at[wid], ridx_v)

    bufs = (buf_a, buf_b)
    sems = (sem_a, sem_b)
    plan = [(node_hbm, hidx_v, c) for c in range(NCHUNK)] + \
           [(relemb_hbm, ridx_v, c) for c in range(NCHUNK)]

    acc = tuple(jnp.zeros((NUM_LANES,), jnp.float32)
                for _ in range(LANE_GROUPS))

    def accumulate(buf, acc):
      def body(r, acc):
        return tuple(acc[j] + buf[r, pl.ds(j * NUM_LANES, NUM_LANES)]
                     for j in range(LANE_GROUPS))
      return lax.fori_loop(0, CHUNK, body, acc)

    table0, idx0, c0 = plan[0]
    handles = [pltpu.async_copy(table0.at[idx0.at[c0]], bufs[0], sems[0])]
    for i in range(len(plan)):
      if i + 1 < len(plan):
        table, idx, c = plan[i + 1]
        handles.append(
            pltpu.async_copy(table.at[idx.at[c]],
                             bufs[(i + 1) % 2], sems[(i + 1) % 2]))
      handles[i].wait()
      acc = accumulate(bufs[i % 2], acc)

    for j in range(LANE_GROUPS):
      acc_v[pl.ds(j * NUM_LANES, NUM_LANES)] = acc[j]
    pltpu.sync_copy(acc_v, out_hbm.at[wid])

  return sc_kernel(head_idx, rel_idx, node_emb, rel_emb)


def _tc_finish(partials, W, b2):
  def body(part_ref, w_ref, b_ref, out_ref):
    pooled = jnp.sum(part_ref[...], axis=0, keepdims=True) * (1.0 / BATCH)
    out_ref[...] = lax.dot_general(
        pooled, w_ref[...], (((1,), (1,)), ((), ())),
        preferred_element_type=jnp.float32) + b_ref[...]

  return pl.pallas_call(
      body,
      out_shape=jax.ShapeDtypeStruct((1, OUT_DIM), jnp.float32),
  )(partials, W, b2)


def kernel(head_index, rel_type, tail_index, node_emb, rel_emb, W, b):
  del tail_index
  h = head_index.astype(jnp.int32).reshape(NW, NCHUNK, CHUNK)
  r = rel_type.astype(jnp.int32).reshape(NW, NCHUNK, CHUNK)
  partials = _sc_partial_sums(h, r, node_emb, rel_emb)
  out = _tc_finish(partials, W, b.reshape(1, OUT_DIM))
  return out.reshape(OUT_DIM)


# speedup vs baseline: 2.5091x; 1.1669x over previous
import functools

import jax
import jax.numpy as jnp
from jax import lax
from jax.experimental import pallas as pl
from jax.experimental.pallas import tpu as pltpu
from jax.experimental.pallas import tpu_sc as plsc

HIDDEN = 256
OUT_DIM = 384
BATCH = 16384
NUM_LANES = 16
LANE_GROUPS = HIDDEN // NUM_LANES
NUM_REL = 1000

NC = 2
NS = 16
NW = NC * NS
B_PER_W = BATCH // NW
CHUNK = 128
NCHUNK = B_PER_W // CHUNK
NBINS = 1024
ROWS_PER_W = NBINS // NS


def _sc_partial_sums(head_idx, rel_idx, node_emb, rel_emb):
  mesh = plsc.VectorSubcoreMesh(core_axis_name="c", subcore_axis_name="s")

  @functools.partial(
      pl.kernel,
      out_type=jax.ShapeDtypeStruct((NW, HIDDEN), jnp.float32),
      mesh=mesh,
      scratch_types=[
          pltpu.VMEM((NCHUNK, CHUNK), jnp.int32),
          pltpu.VMEM((NCHUNK, CHUNK), jnp.int32),
          pltpu.VMEM((CHUNK, HIDDEN), jnp.float32),
          pltpu.VMEM((CHUNK, HIDDEN), jnp.float32),
          pltpu.VMEM((CHUNK, NUM_LANES), jnp.float32),
          pltpu.VMEM((ROWS_PER_W, HIDDEN), jnp.float32),
          pltpu.VMEM((ROWS_PER_W,), jnp.int32),
          pltpu.VMEM((HIDDEN,), jnp.float32),
          pltpu.VMEM_SHARED((NBINS, NUM_LANES), jnp.float32),
          pltpu.SemaphoreType.DMA,
          pltpu.SemaphoreType.DMA,
      ],
  )
  def sc_kernel(head_hbm, rel_hbm, node_hbm, relemb_hbm, out_hbm,
                hidx_v, ridx_v, buf_a, buf_b, stage_v, relrows_v, rid_v,
                acc_v, counts_sh, sem_a, sem_b):
    sid = lax.axis_index("s")
    wid = sid * NC + lax.axis_index("c")
    pltpu.sync_copy(head_hbm.at[wid], hidx_v)
    pltpu.sync_copy(rel_hbm.at[wid], ridx_v)

    bufs = (buf_a, buf_b)
    sems = (sem_a, sem_b)
    handles = [
        pltpu.async_copy(node_hbm.at[hidx_v.at[0]], buf_a, sem_a),
        pltpu.async_copy(node_hbm.at[hidx_v.at[1]], buf_b, sem_b),
    ]

    zeros16 = jnp.zeros((NUM_LANES,), jnp.float32)

    def zero_body(i, _):
      stage_v[i, :] = zeros16
      return 0
    lax.fori_loop(0, ROWS_PER_W, zero_body, 0)
    pltpu.sync_copy(stage_v.at[pl.ds(0, ROWS_PER_W)],
                    counts_sh.at[pl.ds(sid * ROWS_PER_W, ROWS_PER_W)])
    plsc.subcore_barrier()

    ones16 = jnp.ones((NUM_LANES,), jnp.float32)

    def ones_body(i, _):
      stage_v[i, :] = ones16
      return 0
    lax.fori_loop(0, CHUNK, ones_body, 0)
    for c in range(NCHUNK):
      pltpu.sync_copy(stage_v, counts_sh.at[ridx_v.at[c]], add=True)
    plsc.subcore_barrier()

    acc = tuple(jnp.zeros((NUM_LANES,), jnp.float32)
                for _ in range(LANE_GROUPS))

    def accumulate(buf, acc):
      def body(r, acc):
        return tuple(acc[j] + buf[r, pl.ds(j * NUM_LANES, NUM_LANES)]
                     for j in range(LANE_GROUPS))
      return lax.fori_loop(0, CHUNK, body, acc)

    for i in range(NCHUNK):
      handles[i].wait()
      acc = accumulate(bufs[i % 2], acc)
      if i + 2 < NCHUNK:
        handles.append(
            pltpu.async_copy(node_hbm.at[hidx_v.at[i + 2]],
                             bufs[i % 2], sems[i % 2]))

    iota = lax.iota(jnp.int32, NUM_LANES)
    base = sid * ROWS_PER_W
    for k in range(ROWS_PER_W // NUM_LANES):
      ids = base + k * NUM_LANES + iota
      rid_v[pl.ds(k * NUM_LANES, NUM_LANES)] = jnp.minimum(ids, NUM_REL - 1)
    pltpu.async_copy(relemb_hbm.at[rid_v], relrows_v, sem_a).wait()
    pltpu.sync_copy(counts_sh.at[pl.ds(base, ROWS_PER_W)],
                    stage_v.at[pl.ds(0, ROWS_PER_W)])

    def wbody(j, acc):
      cnt = stage_v[j, :]
      return tuple(acc[g] + cnt * relrows_v[j, pl.ds(g * NUM_LANES,
                                                     NUM_LANES)]
                   for g in range(LANE_GROUPS))
    acc = lax.fori_loop(0, ROWS_PER_W, wbody, acc)

    for j in range(LANE_GROUPS):
      acc_v[pl.ds(j * NUM_LANES, NUM_LANES)] = acc[j]
    pltpu.sync_copy(acc_v, out_hbm.at[wid])

  return sc_kernel(head_idx, rel_idx, node_emb, rel_emb)


def _tc_finish(partials, W, b2):
  def body(part_ref, w_ref, b_ref, out_ref):
    pooled = jnp.sum(part_ref[...], axis=0, keepdims=True) * (1.0 / BATCH)
    out_ref[...] = lax.dot_general(
        pooled, w_ref[...], (((1,), (1,)), ((), ())),
        preferred_element_type=jnp.float32) + b_ref[...]

  return pl.pallas_call(
      body,
      out_shape=jax.ShapeDtypeStruct((1, OUT_DIM), jnp.float32),
  )(partials, W, b2)


def kernel(head_index, rel_type, tail_index, node_emb, rel_emb, W, b):
  del tail_index
  h = head_index.astype(jnp.int32).reshape(NW, NCHUNK, CHUNK)
  r = rel_type.astype(jnp.int32).reshape(NW, NCHUNK, CHUNK)
  partials = _sc_partial_sums(h, r, node_emb, rel_emb)
  out = _tc_finish(partials, W, b.reshape(1, OUT_DIM))
  return out.reshape(OUT_DIM)


# speedup vs baseline: 2.7324x; 1.0890x over previous
import functools

import jax
import jax.numpy as jnp
from jax import lax
from jax.experimental import pallas as pl
from jax.experimental.pallas import tpu as pltpu
from jax.experimental.pallas import tpu_sc as plsc

HIDDEN = 256
OUT_DIM = 384
BATCH = 16384
NUM_LANES = 16
LANE_GROUPS = HIDDEN // NUM_LANES
NUM_REL = 1000

NC = 2
NS = 16
NW = NC * NS
B_PER_W = BATCH // NW
CHUNK = 128
NCHUNK = B_PER_W // CHUNK
NBINS = 1024
ROWS_PER_W = NBINS // NS


def _sc_partial_sums(head_idx, rel_idx, node_emb, rel_emb):
  mesh = plsc.VectorSubcoreMesh(core_axis_name="c", subcore_axis_name="s")

  @functools.partial(
      pl.kernel,
      out_type=jax.ShapeDtypeStruct((NW, HIDDEN), jnp.float32),
      mesh=mesh,
      scratch_types=[
          pltpu.VMEM((NCHUNK, CHUNK), jnp.int32),
          pltpu.VMEM((NCHUNK, CHUNK), jnp.int32),
          pltpu.VMEM((CHUNK, HIDDEN), jnp.float32),
          pltpu.VMEM((CHUNK, HIDDEN), jnp.float32),
          pltpu.VMEM((CHUNK, NUM_LANES), jnp.float32),
          pltpu.VMEM((ROWS_PER_W, HIDDEN), jnp.float32),
          pltpu.VMEM((ROWS_PER_W,), jnp.int32),
          pltpu.VMEM((HIDDEN,), jnp.float32),
          pltpu.VMEM_SHARED((NBINS, NUM_LANES), jnp.float32),
          pltpu.SemaphoreType.DMA,
          pltpu.SemaphoreType.DMA,
          pltpu.SemaphoreType.DMA,
      ],
  )
  def sc_kernel(head_hbm, rel_hbm, node_hbm, relemb_hbm, out_hbm,
                hidx_v, ridx_v, buf_a, buf_b, stage_v, relrows_v, rid_v,
                acc_v, counts_sh, sem_a, sem_b, sem_c):
    sid = lax.axis_index("s")
    wid = sid * NC + lax.axis_index("c")
    pltpu.sync_copy(head_hbm.at[wid], hidx_v)
    pltpu.sync_copy(rel_hbm.at[wid], ridx_v)

    bufs = (buf_a, buf_b)
    sems = (sem_a, sem_b)
    handles = [
        pltpu.async_copy(node_hbm.at[hidx_v.at[0]], buf_a, sem_a),
        pltpu.async_copy(node_hbm.at[hidx_v.at[1]], buf_b, sem_b),
    ]

    zeros16 = jnp.zeros((NUM_LANES,), jnp.float32)

    def zero_body(i, _):
      stage_v[i, :] = zeros16
      return 0
    lax.fori_loop(0, ROWS_PER_W, zero_body, 0)
    pltpu.sync_copy(stage_v.at[pl.ds(0, ROWS_PER_W)],
                    counts_sh.at[pl.ds(sid * ROWS_PER_W, ROWS_PER_W)])
    plsc.subcore_barrier()

    ones16 = jnp.ones((NUM_LANES,), jnp.float32)

    def ones_body(i, _):
      stage_v[i, :] = ones16
      return 0
    lax.fori_loop(0, CHUNK, ones_body, 0)
    for c in range(NCHUNK):
      pltpu.sync_copy(stage_v, counts_sh.at[ridx_v.at[c]], add=True)
    plsc.subcore_barrier()

    iota = lax.iota(jnp.int32, NUM_LANES)
    base = sid * ROWS_PER_W
    for k in range(ROWS_PER_W // NUM_LANES):
      ids = base + k * NUM_LANES + iota
      rid_v[pl.ds(k * NUM_LANES, NUM_LANES)] = jnp.minimum(ids, NUM_REL - 1)
    rel_handle = pltpu.async_copy(relemb_hbm.at[rid_v], relrows_v, sem_c)
    pltpu.sync_copy(counts_sh.at[pl.ds(base, ROWS_PER_W)],
                    stage_v.at[pl.ds(0, ROWS_PER_W)])

    acc = tuple(jnp.zeros((NUM_LANES,), jnp.float32)
                for _ in range(LANE_GROUPS))

    def accumulate(buf, acc):
      def body(r, acc):
        return tuple(acc[j] + buf[r, pl.ds(j * NUM_LANES, NUM_LANES)]
                     for j in range(LANE_GROUPS))
      return lax.fori_loop(0, CHUNK, body, acc)

    for i in range(NCHUNK):
      handles[i].wait()
      acc = accumulate(bufs[i % 2], acc)
      if i + 2 < NCHUNK:
        handles.append(
            pltpu.async_copy(node_hbm.at[hidx_v.at[i + 2]],
                             bufs[i % 2], sems[i % 2]))

    rel_handle.wait()

    def wbody(j, acc):
      cnt = stage_v[j, :]
      return tuple(acc[g] + cnt * relrows_v[j, pl.ds(g * NUM_LANES,
                                                     NUM_LANES)]
                   for g in range(LANE_GROUPS))
    acc = lax.fori_loop(0, ROWS_PER_W, wbody, acc)

    for j in range(LANE_GROUPS):
      acc_v[pl.ds(j * NUM_LANES, NUM_LANES)] = acc[j]
    pltpu.sync_copy(acc_v, out_hbm.at[wid])

  return sc_kernel(head_idx, rel_idx, node_emb, rel_emb)


def _tc_finish(partials, W, b):
  def body(part_ref, w_ref, b_ref, out_ref):
    pooled = jnp.sum(part_ref[...], axis=0, keepdims=True) * (1.0 / BATCH)
    res = lax.dot_general(
        pooled, w_ref[...], (((1,), (1,)), ((), ())),
        preferred_element_type=jnp.float32)
    out_ref[...] = res[0] + b_ref[...]

  return pl.pallas_call(
      body,
      out_shape=jax.ShapeDtypeStruct((OUT_DIM,), jnp.float32),
  )(partials, W, b)


def kernel(head_index, rel_type, tail_index, node_emb, rel_emb, W, b):
  del tail_index
  h = head_index.astype(jnp.int32).reshape(NW, NCHUNK, CHUNK)
  r = rel_type.astype(jnp.int32).reshape(NW, NCHUNK, CHUNK)
  partials = _sc_partial_sums(h, r, node_emb, rel_emb)
  return _tc_finish(partials, W, b)
